# speedup vs baseline: 1.0003x; 1.0003x over previous
import jax
import jax.numpy as jnp
import numpy as np
from jax.experimental import pallas as pl
from jax.experimental.pallas import tpu as pltpu

N = 10000
E = 320000
D = 128
B = 256
H = 8
DH = D // H
K1 = 75


def _gcn_conv(x, src, dst, w, b, dinv):
    xw = x @ w
    norm = dinv[src] * dinv[dst]
    msg = xw[src] * norm[:, None]
    agg = jax.ops.segment_sum(msg, dst, num_segments=N) + xw * (dinv * dinv)[:, None]
    return agg + b


def _pma_graph(xn, src, dst, batch, p, dinv):
    q = p['S1'] @ p['p1_q_w'] + p['p1_q_b']
    k = _gcn_conv(xn, src, dst, p['p1_k_w'], p['p1_k_b'], dinv)
    v = _gcn_conv(xn, src, dst, p['p1_v_w'], p['p1_v_b'], dinv)
    qh = q.reshape(K1, H, DH)
    kh = k.reshape(N, H, DH)
    vh = v.reshape(N, H, DH)
    scores = jnp.einsum('nhd,shd->nsh', kh, qh) / np.sqrt(D)
    m = jax.ops.segment_max(scores, batch, num_segments=B)
    e = jnp.exp(scores - m[batch])
    denom = jax.ops.segment_sum(e, batch, num_segments=B)
    a = e / jnp.maximum(denom[batch], 1e-12)
    outs = []
    for h in range(H):
        contrib = a[:, :, h, None] * vh[:, None, h, :]
        outs.append(jax.ops.segment_sum(contrib, batch, num_segments=B))
    att = jnp.stack(outs, axis=2).reshape(B, K1, D)
    o = q[None] + att
    o = o + jax.nn.relu(o @ p['p1_o_w'] + p['p1_o_b'])
    return o


def _mab_dense(q_in, kv_in, p, prefix):
    q = q_in @ p[prefix + '_q_w'] + p[prefix + '_q_b']
    k = kv_in @ p[prefix + '_k_w'] + p[prefix + '_k_b']
    v = kv_in @ p[prefix + '_v_w'] + p[prefix + '_v_b']
    b, s = q.shape[0], q.shape[1]
    n = k.shape[1]
    qh = q.reshape(b, s, H, DH)
    kh = k.reshape(b, n, H, DH)
    vh = v.reshape(b, n, H, DH)
    scores = jnp.einsum('bshd,bnhd->bhsn', qh, kh) / np.sqrt(D)
    a = jax.nn.softmax(scores, axis=-1)
    att = jnp.einsum('bhsn,bnhd->bshd', a, vh).reshape(b, s, D)
    o = q + att
    o = o + jax.nn.relu(o @ p[prefix + '_o_w'] + p[prefix + '_o_b'])
    return o


def _tail_kernel(x_ref, w2_ref, b2_ref, wg_ref, bg_ref, o_ref):
    h = jnp.dot(x_ref[...], w2_ref[...], preferred_element_type=jnp.float32)
    h = h + b2_ref[...]
    g = jnp.dot(h, wg_ref[...], preferred_element_type=jnp.float32) + bg_ref[...]
    o_ref[...] = jax.nn.sigmoid(g)


def kernel(herg_em, x, edge_index, edge_attr, batch, mol_rep, params):
    p = params
    src, dst = edge_index[0], edge_index[1]
    deg = jax.ops.segment_sum(jnp.ones((E,), jnp.float32), dst, num_segments=N) + 1.0
    dinv = jax.lax.rsqrt(deg)
    h = jax.nn.relu(_gcn_conv(x, src, dst, p['conv0_w'], p['conv0_b'], dinv))
    for i in (1, 2, 3):
        h = jax.nn.relu(_gcn_conv(h, src, dst, p['conv%d_w' % i], p['conv%d_b' % i], dinv))
    xn = h @ p['lin1_w'] + p['lin1_b']
    X = _pma_graph(xn, src, dst, batch, p, dinv)
    X = _mab_dense(X, X, p, 'sab')
    X = _mab_dense(jnp.broadcast_to(p['S2'][None], (B, 1, D)), X, p, 'p2')
    X = X[:, 0, :]
    out = pl.pallas_call(
        _tail_kernel,
        out_shape=jax.ShapeDtypeStruct((B, 1), jnp.float32),
    )(X, p['lin2_w'], p['lin2_b'], p['gout_w'], p['gout_b'])
    return out


# speedup vs baseline: 2.2889x; 2.2882x over previous
import functools

import jax
import jax.numpy as jnp
import numpy as np
from jax import lax
from jax.experimental import pallas as pl
from jax.experimental.pallas import tpu as pltpu
from jax.experimental.pallas import tpu_sc as plsc

N = 10000
E = 320000
D = 128
B = 256
H = 8
DH = D // H
K1 = 75

_NC, _NS = 2, 16
_NW = _NC * _NS
_NPAD = N + 240
_RPS = _NPAD // _NS
_NKC = (E + _NW * 128 - 1) // (_NW * 128)
_EPAD = _NW * _NKC * 128

_mesh = plsc.VectorSubcoreMesh(core_axis_name="c", subcore_axis_name="s")


def _make_row_scatter(dcols):
    @functools.partial(
        pl.kernel,
        out_type=jax.ShapeDtypeStruct((_NC, _NPAD, dcols), jnp.float32),
        mesh=_mesh,
        scratch_types=[
            pltpu.VMEM((_NKC, 128), jnp.int32),
            pltpu.VMEM((_NKC, 128), jnp.int32),
            pltpu.VMEM((128, dcols), jnp.float32),
            pltpu.VMEM_SHARED((_NPAD, dcols), jnp.float32),
            pltpu.SemaphoreType.DMA,
        ],
    )
    def pass_k(y_hbm, srcw, dstw, zeros_hbm, out_hbm, src_v, dst_v, rows_v, acc_sh, sem):
        cid = lax.axis_index("c")
        sid = lax.axis_index("s")
        wid = sid * _NC + cid
        pltpu.sync_copy(srcw.at[wid], src_v)
        pltpu.sync_copy(dstw.at[wid], dst_v)
        pltpu.sync_copy(zeros_hbm.at[pl.ds(sid * _RPS, _RPS)],
                        acc_sh.at[pl.ds(sid * _RPS, _RPS)])
        plsc.subcore_barrier()

        def body(j, carry):
            pltpu.async_copy(y_hbm.at[src_v.at[j]], rows_v, sem).wait()
            pltpu.sync_copy(rows_v, acc_sh.at[dst_v.at[j]], add=True)
            return carry

        lax.fori_loop(0, _NKC, body, 0)
        plsc.subcore_barrier()
        pltpu.sync_copy(acc_sh.at[pl.ds(sid * _RPS, _RPS)],
                        out_hbm.at[cid, pl.ds(sid * _RPS, _RPS)])

    return pass_k


_row_scatter_128 = _make_row_scatter(D)


def _pad_rows(y):
    return jnp.concatenate([y, jnp.zeros((_NPAD - N, y.shape[1]), y.dtype)], axis=0)


def _sc_segment_sum(y, srcw, dstw, zeros):
    out = _row_scatter_128(_pad_rows(y), srcw, dstw, zeros)
    return out[0, :N] + out[1, :N]


def _conv_apply(x, w, b, dinv, srcw, dstw, zeros):
    y = dinv[:, None] * (x @ w)
    s = _sc_segment_sum(y, srcw, dstw, zeros)
    return dinv[:, None] * (s + y) + b


def _pma_graph(xn, batch, p, dinv, srcw, dstw, zeros):
    q = p['S1'] @ p['p1_q_w'] + p['p1_q_b']
    k = _conv_apply(xn, p['p1_k_w'], p['p1_k_b'], dinv, srcw, dstw, zeros)
    v = _conv_apply(xn, p['p1_v_w'], p['p1_v_b'], dinv, srcw, dstw, zeros)
    qh = q.reshape(K1, H, DH)
    kh = k.reshape(N, H, DH)
    vh = v.reshape(N, H, DH)
    scores = jnp.einsum('nhd,shd->nsh', kh, qh) / np.sqrt(D)
    m = jax.ops.segment_max(scores, batch, num_segments=B)
    e = jnp.exp(scores - m[batch])
    denom = jax.ops.segment_sum(e, batch, num_segments=B)
    a = e / jnp.maximum(denom[batch], 1e-12)
    outs = []
    for h in range(H):
        contrib = a[:, :, h, None] * vh[:, None, h, :]
        outs.append(jax.ops.segment_sum(contrib, batch, num_segments=B))
    att = jnp.stack(outs, axis=2).reshape(B, K1, D)
    o = q[None] + att
    o = o + jax.nn.relu(o @ p['p1_o_w'] + p['p1_o_b'])
    return o


def _mab_dense(q_in, kv_in, p, prefix):
    q = q_in @ p[prefix + '_q_w'] + p[prefix + '_q_b']
    k = kv_in @ p[prefix + '_k_w'] + p[prefix + '_k_b']
    v = kv_in @ p[prefix + '_v_w'] + p[prefix + '_v_b']
    b, s = q.shape[0], q.shape[1]
    n = k.shape[1]
    qh = q.reshape(b, s, H, DH)
    kh = k.reshape(b, n, H, DH)
    vh = v.reshape(b, n, H, DH)
    scores = jnp.einsum('bshd,bnhd->bhsn', qh, kh) / np.sqrt(D)
    a = jax.nn.softmax(scores, axis=-1)
    att = jnp.einsum('bhsn,bnhd->bshd', a, vh).reshape(b, s, D)
    o = q + att
    o = o + jax.nn.relu(o @ p[prefix + '_o_w'] + p[prefix + '_o_b'])
    return o


def _tail_kernel(x_ref, w2_ref, b2_ref, wg_ref, bg_ref, o_ref):
    h = jnp.dot(x_ref[...], w2_ref[...], preferred_element_type=jnp.float32)
    h = h + b2_ref[...]
    g = jnp.dot(h, wg_ref[...], preferred_element_type=jnp.float32) + bg_ref[...]
    o_ref[...] = jax.nn.sigmoid(g)


def kernel(herg_em, x, edge_index, edge_attr, batch, mol_rep, params):
    p = params
    src = edge_index[0].astype(jnp.int32)
    dst = edge_index[1].astype(jnp.int32)
    pad_idx = (N + jnp.arange(_EPAD - E, dtype=jnp.int32) % (_NPAD - N))
    srcw = jnp.concatenate([src, pad_idx]).reshape(_NW, _NKC, 128)
    dstw = jnp.concatenate([dst, pad_idx]).reshape(_NW, _NKC, 128)
    zeros = jnp.zeros((_NPAD, D), jnp.float32)
    ones_table = jnp.concatenate(
        [jnp.ones((N, D), jnp.float32), jnp.zeros((_NPAD - N, D), jnp.float32)], axis=0)

    degp = _row_scatter_128(ones_table, srcw, dstw, zeros)
    deg = degp[0, :N, 0] + degp[1, :N, 0] + 1.0
    dinv = lax.rsqrt(deg)

    h = jax.nn.relu(_conv_apply(x, p['conv0_w'], p['conv0_b'], dinv, srcw, dstw, zeros))
    for i in (1, 2, 3):
        h = jax.nn.relu(_conv_apply(h, p['conv%d_w' % i], p['conv%d_b' % i],
                                    dinv, srcw, dstw, zeros))
    xn = h @ p['lin1_w'] + p['lin1_b']
    X = _pma_graph(xn, batch, p, dinv, srcw, dstw, zeros)
    X = _mab_dense(X, X, p, 'sab')
    X = _mab_dense(jnp.broadcast_to(p['S2'][None], (B, 1, D)), X, p, 'p2')
    X = X[:, 0, :]
    out = pl.pallas_call(
        _tail_kernel,
        out_shape=jax.ShapeDtypeStruct((B, 1), jnp.float32),
    )(X, p['lin2_w'], p['lin2_b'], p['gout_w'], p['gout_b'])
    return out


# speedup vs baseline: 11.8113x; 5.1602x over previous
import functools

import jax
import jax.numpy as jnp
import numpy as np
from jax import lax
from jax.experimental import pallas as pl
from jax.experimental.pallas import tpu as pltpu
from jax.experimental.pallas import tpu_sc as plsc

N = 10000
E = 320000
D = 128
B = 256
H = 8
DH = D // H
K1 = 75

_NC, _NS = 2, 16
_NW = _NC * _NS
_NPAD = N + 240
_RPS = _NPAD // _NS
_NKC = (E + _NW * 128 - 1) // (_NW * 128)
_EPAD = _NW * _NKC * 128

@functools.lru_cache(maxsize=None)
def _make_row_scatter(dcols):
    mesh = plsc.VectorSubcoreMesh(core_axis_name="c", subcore_axis_name="s")

    @functools.partial(
        pl.kernel,
        out_type=jax.ShapeDtypeStruct((_NC, _NPAD, dcols), jnp.float32),
        mesh=mesh,
        scratch_types=[
            pltpu.VMEM((_NKC, 128), jnp.int32),
            pltpu.VMEM((_NKC, 128), jnp.int32),
            pltpu.VMEM((128, dcols), jnp.float32),
            pltpu.VMEM_SHARED((_NPAD, dcols), jnp.float32),
            pltpu.SemaphoreType.DMA,
        ],
    )
    def pass_k(y_hbm, srcw, dstw, zeros_hbm, out_hbm, src_v, dst_v, rows_v, acc_sh, sem):
        cid = lax.axis_index("c")
        sid = lax.axis_index("s")
        wid = sid * _NC + cid
        pltpu.sync_copy(srcw.at[wid], src_v)
        pltpu.sync_copy(dstw.at[wid], dst_v)
        pltpu.sync_copy(zeros_hbm.at[pl.ds(sid * _RPS, _RPS)],
                        acc_sh.at[pl.ds(sid * _RPS, _RPS)])
        plsc.subcore_barrier()

        def body(j, carry):
            pltpu.async_copy(y_hbm.at[src_v.at[j]], rows_v, sem).wait()
            pltpu.sync_copy(rows_v, acc_sh.at[dst_v.at[j]], add=True)
            return carry

        lax.fori_loop(0, _NKC, body, 0)
        plsc.subcore_barrier()
        pltpu.sync_copy(acc_sh.at[pl.ds(sid * _RPS, _RPS)],
                        out_hbm.at[cid, pl.ds(sid * _RPS, _RPS)])

    return pass_k


def _pad_rows(y):
    return jnp.concatenate([y, jnp.zeros((_NPAD - N, y.shape[1]), y.dtype)], axis=0)


def _sc_segment_sum(y, srcw, dstw, zeros):
    out = _make_row_scatter(D)(_pad_rows(y), srcw, dstw, zeros)
    return out[0, :N] + out[1, :N]


def _conv_apply(x, w, b, dinv, srcw, dstw, zeros):
    y = dinv[:, None] * (x @ w)
    s = _sc_segment_sum(y, srcw, dstw, zeros)
    return dinv[:, None] * (s + y) + b


_SP = 80


def _pma_att_kernel(offs_ref, k_ref, v_ref, qb_ref, o_ref):
    b = pl.program_id(0)
    start = offs_ref[b]
    end = offs_ref[b + 1]
    c0 = start // 128
    c1 = lax.div(end + 127, 128)

    def body(c, carry):
        m, l, accs = carry
        base = c * 128
        rows = k_ref[pl.ds(base * 1, 128), :]
        vrows = v_ref[pl.ds(base * 1, 128), :]
        ids = base + lax.broadcasted_iota(jnp.int32, (128, 1), 0)
        valid = (ids >= start) & (ids < end)
        s = jnp.dot(rows, qb_ref[...], preferred_element_type=jnp.float32)
        s = jnp.where(valid, s, -1e30)
        m_new = jnp.maximum(m, jnp.max(s, axis=0, keepdims=True))
        corr = jnp.exp(m - m_new)
        e = jnp.exp(s - m_new)
        e = jnp.where(valid, e, 0.0)
        l = l * corr + jnp.sum(e, axis=0, keepdims=True)
        new_accs = []
        for h in range(H):
            eh = e[:, h * _SP:(h + 1) * _SP]
            vh = vrows[:, h * DH:(h + 1) * DH]
            ch = corr[0, h * _SP:(h + 1) * _SP]
            prod = lax.dot_general(eh, vh, (((0,), (0,)), ((), ())),
                                   preferred_element_type=jnp.float32)
            new_accs.append(accs[h] * ch[:, None] + prod)
        return m_new, l, tuple(new_accs)

    m0 = jnp.full((1, H * _SP), -1e30, jnp.float32)
    l0 = jnp.zeros((1, H * _SP), jnp.float32)
    a0 = tuple(jnp.zeros((_SP, DH), jnp.float32) for _ in range(H))
    m, l, accs = lax.fori_loop(c0, c1, body, (m0, l0, a0))
    outs = []
    for h in range(H):
        lh = jnp.maximum(l[0, h * _SP:(h + 1) * _SP], 1e-12)
        outs.append(accs[h] / lh[:, None])
    o_ref[0] = jnp.concatenate(outs, axis=1)


def _pma_attention(q, k, v, offs):
    qh = q.reshape(K1, H, DH)
    qb = jnp.zeros((H, DH, H, _SP), jnp.float32)
    qb = qb.at[jnp.arange(H), :, jnp.arange(H), :K1].set(
        jnp.transpose(qh, (1, 2, 0)) / np.sqrt(D))
    qb = qb.reshape(D, H * _SP)
    kp = _pad_rows(k)
    vp = _pad_rows(v)
    att = pl.pallas_call(
        _pma_att_kernel,
        grid=(B,),
        in_specs=[
            pl.BlockSpec(memory_space=pltpu.SMEM),
            pl.BlockSpec((_NPAD, D), lambda b: (0, 0)),
            pl.BlockSpec((_NPAD, D), lambda b: (0, 0)),
            pl.BlockSpec((D, H * _SP), lambda b: (0, 0)),
        ],
        out_specs=pl.BlockSpec((1, _SP, D), lambda b: (b, 0, 0)),
        out_shape=jax.ShapeDtypeStruct((B, _SP, D), jnp.float32),
    )(offs, kp, vp, qb)
    return att[:, :K1, :]


def _pma_graph(xn, batch, offs, p, dinv, srcw, dstw, zeros):
    q = p['S1'] @ p['p1_q_w'] + p['p1_q_b']
    k = _conv_apply(xn, p['p1_k_w'], p['p1_k_b'], dinv, srcw, dstw, zeros)
    v = _conv_apply(xn, p['p1_v_w'], p['p1_v_b'], dinv, srcw, dstw, zeros)
    att = _pma_attention(q, k, v, offs)
    o = q[None] + att
    o = o + jax.nn.relu(o @ p['p1_o_w'] + p['p1_o_b'])
    return o


def _mab_dense(q_in, kv_in, p, prefix):
    q = q_in @ p[prefix + '_q_w'] + p[prefix + '_q_b']
    k = kv_in @ p[prefix + '_k_w'] + p[prefix + '_k_b']
    v = kv_in @ p[prefix + '_v_w'] + p[prefix + '_v_b']
    b, s = q.shape[0], q.shape[1]
    n = k.shape[1]
    qh = q.reshape(b, s, H, DH)
    kh = k.reshape(b, n, H, DH)
    vh = v.reshape(b, n, H, DH)
    scores = jnp.einsum('bshd,bnhd->bhsn', qh, kh) / np.sqrt(D)
    a = jax.nn.softmax(scores, axis=-1)
    att = jnp.einsum('bhsn,bnhd->bshd', a, vh).reshape(b, s, D)
    o = q + att
    o = o + jax.nn.relu(o @ p[prefix + '_o_w'] + p[prefix + '_o_b'])
    return o


def _tail_kernel(x_ref, w2_ref, b2_ref, wg_ref, bg_ref, o_ref):
    h = jnp.dot(x_ref[...], w2_ref[...], preferred_element_type=jnp.float32)
    h = h + b2_ref[...]
    g = jnp.dot(h, wg_ref[...], preferred_element_type=jnp.float32) + bg_ref[...]
    o_ref[...] = jax.nn.sigmoid(g)


def kernel(herg_em, x, edge_index, edge_attr, batch, mol_rep, params):
    p = params
    src = edge_index[0].astype(jnp.int32)
    dst = edge_index[1].astype(jnp.int32)
    pad_idx = (N + jnp.arange(_EPAD - E, dtype=jnp.int32) % (_NPAD - N))
    srcw = jnp.concatenate([src, pad_idx]).reshape(_NW, _NKC, 128)
    dstw = jnp.concatenate([dst, pad_idx]).reshape(_NW, _NKC, 128)
    zeros = jnp.zeros((_NPAD, D), jnp.float32)
    ones_table = jnp.concatenate(
        [jnp.ones((N, D), jnp.float32), jnp.zeros((_NPAD - N, D), jnp.float32)], axis=0)

    degp = _make_row_scatter(D)(ones_table, srcw, dstw, zeros)
    deg = degp[0, :N, 0] + degp[1, :N, 0] + 1.0
    dinv = lax.rsqrt(deg)

    h = jax.nn.relu(_conv_apply(x, p['conv0_w'], p['conv0_b'], dinv, srcw, dstw, zeros))
    for i in (1, 2, 3):
        h = jax.nn.relu(_conv_apply(h, p['conv%d_w' % i], p['conv%d_b' % i],
                                    dinv, srcw, dstw, zeros))
    xn = h @ p['lin1_w'] + p['lin1_b']
    offs = jnp.searchsorted(batch.astype(jnp.int32), jnp.arange(B + 1, dtype=jnp.int32)).astype(jnp.int32)
    X = _pma_graph(xn, batch, offs, p, dinv, srcw, dstw, zeros)
    X = _mab_dense(X, X, p, 'sab')
    X = _mab_dense(jnp.broadcast_to(p['S2'][None], (B, 1, D)), X, p, 'p2')
    X = X[:, 0, :]
    out = pl.pallas_call(
        _tail_kernel,
        out_shape=jax.ShapeDtypeStruct((B, 1), jnp.float32),
    )(X, p['lin2_w'], p['lin2_b'], p['gout_w'], p['gout_b'])
    return out


# speedup vs baseline: 13.0005x; 1.1007x over previous
import functools

import jax
import jax.numpy as jnp
import numpy as np
from jax import lax
from jax.experimental import pallas as pl
from jax.experimental.pallas import tpu as pltpu
from jax.experimental.pallas import tpu_sc as plsc

N = 10000
E = 320000
D = 128
B = 256
H = 8
DH = D // H
K1 = 75

_NC, _NS = 2, 16
_NW = _NC * _NS
_NPAD = N + 240
_RPS = _NPAD // _NS
_NKC = 80
_EPAD = _NW * _NKC * 128

_UNROLL = 2


@functools.lru_cache(maxsize=None)
def _make_row_scatter(dcols):
    mesh = plsc.VectorSubcoreMesh(core_axis_name="c", subcore_axis_name="s")

    @functools.partial(
        pl.kernel,
        out_type=jax.ShapeDtypeStruct((_NC, _NPAD, dcols), jnp.float32),
        mesh=mesh,
        scratch_types=[
            pltpu.VMEM((_NKC // 2, 128), jnp.int32),
            pltpu.VMEM((_NKC // 2, 128), jnp.int32),
            pltpu.VMEM((_UNROLL, 128, dcols), jnp.float32),
            pltpu.VMEM_SHARED((_NPAD, dcols), jnp.float32),
        ] + [pltpu.SemaphoreType.DMA] * (2 * _UNROLL),
    )
    def pass_k(y_hbm, srcw, dstw, zeros_hbm, out_hbm, src_v, dst_v, rows_v, acc_sh, *sems):
        gsem = sems[:_UNROLL]
        ssem = sems[_UNROLL:]
        cid = lax.axis_index("c")
        sid = lax.axis_index("s")
        wid = sid * _NC + cid
        pltpu.sync_copy(zeros_hbm.at[pl.ds(sid * _RPS, _RPS)],
                        acc_sh.at[pl.ds(sid * _RPS, _RPS)])
        plsc.subcore_barrier()

        def body(i, carry):
            j = i * _UNROLL
            gathers = []
            for u in range(_UNROLL):
                gathers.append(pltpu.async_copy(
                    y_hbm.at[src_v.at[j + u]], rows_v.at[u], gsem[u]))
            scatters = []
            for u in range(_UNROLL):
                gathers[u].wait()
                scatters.append(pltpu.async_copy(
                    rows_v.at[u], acc_sh.at[dst_v.at[j + u]], ssem[u], add=True))
            for u in range(_UNROLL):
                scatters[u].wait()
            return carry

        half = _NKC // 2
        for ph in range(2):
            pltpu.sync_copy(srcw.at[wid, pl.ds(ph * half, half)], src_v)
            pltpu.sync_copy(dstw.at[wid, pl.ds(ph * half, half)], dst_v)
            lax.fori_loop(0, half // _UNROLL, body, 0)
        plsc.subcore_barrier()
        pltpu.sync_copy(acc_sh.at[pl.ds(sid * _RPS, _RPS)],
                        out_hbm.at[cid, pl.ds(sid * _RPS, _RPS)])

    return pass_k


def _pad_rows(y):
    return jnp.concatenate([y, jnp.zeros((_NPAD - N, y.shape[1]), y.dtype)], axis=0)


def _sc_segment_sum(y, srcw, dstw, zeros):
    out = _make_row_scatter(D)(_pad_rows(y), srcw, dstw, zeros)
    return out[0, :N] + out[1, :N]


def _conv_apply(x, w, b, dinv, srcw, dstw, zeros):
    y = dinv[:, None] * (x @ w)
    s = _sc_segment_sum(y, srcw, dstw, zeros)
    return dinv[:, None] * (s + y) + b


_SP = 80


def _pma_att_kernel(offs_ref, k_ref, v_ref, qb_ref, o_ref):
    b = pl.program_id(0)
    start = offs_ref[b]
    end = offs_ref[b + 1]
    c0 = start // 128
    c1 = lax.div(end + 127, 128)

    def body(c, carry):
        m, l, accs = carry
        base = c * 128
        rows = k_ref[pl.ds(base * 1, 128), :]
        vrows = v_ref[pl.ds(base * 1, 128), :]
        ids = base + lax.broadcasted_iota(jnp.int32, (128, 1), 0)
        valid = (ids >= start) & (ids < end)
        s = jnp.dot(rows, qb_ref[...], preferred_element_type=jnp.float32)
        s = jnp.where(valid, s, -1e30)
        m_new = jnp.maximum(m, jnp.max(s, axis=0, keepdims=True))
        corr = jnp.exp(m - m_new)
        e = jnp.exp(s - m_new)
        e = jnp.where(valid, e, 0.0)
        l = l * corr + jnp.sum(e, axis=0, keepdims=True)
        new_accs = []
        for h in range(H):
            eh = e[:, h * _SP:(h + 1) * _SP]
            vh = vrows[:, h * DH:(h + 1) * DH]
            ch = corr[0, h * _SP:(h + 1) * _SP]
            prod = lax.dot_general(eh, vh, (((0,), (0,)), ((), ())),
                                   preferred_element_type=jnp.float32)
            new_accs.append(accs[h] * ch[:, None] + prod)
        return m_new, l, tuple(new_accs)

    m0 = jnp.full((1, H * _SP), -1e30, jnp.float32)
    l0 = jnp.zeros((1, H * _SP), jnp.float32)
    a0 = tuple(jnp.zeros((_SP, DH), jnp.float32) for _ in range(H))
    m, l, accs = lax.fori_loop(c0, c1, body, (m0, l0, a0))
    outs = []
    for h in range(H):
        lh = jnp.maximum(l[0, h * _SP:(h + 1) * _SP], 1e-12)
        outs.append(accs[h] / lh[:, None])
    o_ref[0] = jnp.concatenate(outs, axis=1)


def _pma_attention(q, k, v, offs):
    qh = q.reshape(K1, H, DH)
    qb = jnp.zeros((H, DH, H, _SP), jnp.float32)
    qb = qb.at[jnp.arange(H), :, jnp.arange(H), :K1].set(
        jnp.transpose(qh, (1, 2, 0)) / np.sqrt(D))
    qb = qb.reshape(D, H * _SP)
    kp = _pad_rows(k)
    vp = _pad_rows(v)
    att = pl.pallas_call(
        _pma_att_kernel,
        grid=(B,),
        in_specs=[
            pl.BlockSpec(memory_space=pltpu.SMEM),
            pl.BlockSpec((_NPAD, D), lambda b: (0, 0)),
            pl.BlockSpec((_NPAD, D), lambda b: (0, 0)),
            pl.BlockSpec((D, H * _SP), lambda b: (0, 0)),
        ],
        out_specs=pl.BlockSpec((1, _SP, D), lambda b: (b, 0, 0)),
        out_shape=jax.ShapeDtypeStruct((B, _SP, D), jnp.float32),
    )(offs, kp, vp, qb)
    return att[:, :K1, :]


def _pma_graph(xn, batch, offs, p, dinv, srcw, dstw, zeros):
    q = p['S1'] @ p['p1_q_w'] + p['p1_q_b']
    k = _conv_apply(xn, p['p1_k_w'], p['p1_k_b'], dinv, srcw, dstw, zeros)
    v = _conv_apply(xn, p['p1_v_w'], p['p1_v_b'], dinv, srcw, dstw, zeros)
    att = _pma_attention(q, k, v, offs)
    o = q[None] + att
    o = o + jax.nn.relu(o @ p['p1_o_w'] + p['p1_o_b'])
    return o


def _mab_dense(q_in, kv_in, p, prefix):
    q = q_in @ p[prefix + '_q_w'] + p[prefix + '_q_b']
    k = kv_in @ p[prefix + '_k_w'] + p[prefix + '_k_b']
    v = kv_in @ p[prefix + '_v_w'] + p[prefix + '_v_b']
    b, s = q.shape[0], q.shape[1]
    n = k.shape[1]
    qh = q.reshape(b, s, H, DH)
    kh = k.reshape(b, n, H, DH)
    vh = v.reshape(b, n, H, DH)
    scores = jnp.einsum('bshd,bnhd->bhsn', qh, kh) / np.sqrt(D)
    a = jax.nn.softmax(scores, axis=-1)
    att = jnp.einsum('bhsn,bnhd->bshd', a, vh).reshape(b, s, D)
    o = q + att
    o = o + jax.nn.relu(o @ p[prefix + '_o_w'] + p[prefix + '_o_b'])
    return o


def _tail_kernel(x_ref, w2_ref, b2_ref, wg_ref, bg_ref, o_ref):
    h = jnp.dot(x_ref[...], w2_ref[...], preferred_element_type=jnp.float32)
    h = h + b2_ref[...]
    g = jnp.dot(h, wg_ref[...], preferred_element_type=jnp.float32) + bg_ref[...]
    o_ref[...] = jax.nn.sigmoid(g)


def kernel(herg_em, x, edge_index, edge_attr, batch, mol_rep, params):
    p = params
    src = edge_index[0].astype(jnp.int32)
    dst = edge_index[1].astype(jnp.int32)
    pad_idx = (N + jnp.arange(_EPAD - E, dtype=jnp.int32) % (_NPAD - N))
    srcw = jnp.concatenate([src, pad_idx]).reshape(_NW, _NKC, 128)
    dstw = jnp.concatenate([dst, pad_idx]).reshape(_NW, _NKC, 128)
    zeros = jnp.zeros((_NPAD, D), jnp.float32)
    ones_table = jnp.concatenate(
        [jnp.ones((N, D), jnp.float32), jnp.zeros((_NPAD - N, D), jnp.float32)], axis=0)

    degp = _make_row_scatter(D)(ones_table, srcw, dstw, zeros)
    deg = degp[0, :N, 0] + degp[1, :N, 0] + 1.0
    dinv = lax.rsqrt(deg)

    h = jax.nn.relu(_conv_apply(x, p['conv0_w'], p['conv0_b'], dinv, srcw, dstw, zeros))
    for i in (1, 2, 3):
        h = jax.nn.relu(_conv_apply(h, p['conv%d_w' % i], p['conv%d_b' % i],
                                    dinv, srcw, dstw, zeros))
    xn = h @ p['lin1_w'] + p['lin1_b']
    offs = jnp.searchsorted(batch.astype(jnp.int32), jnp.arange(B + 1, dtype=jnp.int32)).astype(jnp.int32)
    X = _pma_graph(xn, batch, offs, p, dinv, srcw, dstw, zeros)
    X = _mab_dense(X, X, p, 'sab')
    X = _mab_dense(jnp.broadcast_to(p['S2'][None], (B, 1, D)), X, p, 'p2')
    X = X[:, 0, :]
    out = pl.pallas_call(
        _tail_kernel,
        out_shape=jax.ShapeDtypeStruct((B, 1), jnp.float32),
    )(X, p['lin2_w'], p['lin2_b'], p['gout_w'], p['gout_b'])
    return out


# speedup vs baseline: 13.0450x; 1.0034x over previous
import functools

import jax
import jax.numpy as jnp
import numpy as np
from jax import lax
from jax.experimental import pallas as pl
from jax.experimental.pallas import tpu as pltpu
from jax.experimental.pallas import tpu_sc as plsc

N = 10000
E = 320000
D = 128
B = 256
H = 8
DH = D // H
K1 = 75

_NC, _NS = 2, 16
_NW = _NC * _NS
_NPAD = N + 240
_RPS = _NPAD // _NS
_NKC = 80
_EPAD = _NW * _NKC * 128

_UNROLL = 2


@functools.lru_cache(maxsize=None)
def _make_row_scatter(dcols):
    mesh = plsc.VectorSubcoreMesh(core_axis_name="c", subcore_axis_name="s")

    @functools.partial(
        pl.kernel,
        out_type=jax.ShapeDtypeStruct((_NC, _NPAD, dcols), jnp.float32),
        mesh=mesh,
        scratch_types=[
            pltpu.VMEM((_NKC // 2, 128), jnp.int32),
            pltpu.VMEM((_NKC // 2, 128), jnp.int32),
            pltpu.VMEM((_UNROLL, 128, dcols), jnp.float32),
            pltpu.VMEM_SHARED((_NPAD, dcols), jnp.float32),
        ] + [pltpu.SemaphoreType.DMA] * (2 * _UNROLL),
    )
    def pass_k(y_hbm, srcw, dstw, zeros_hbm, out_hbm, src_v, dst_v, rows_v, acc_sh, *sems):
        gsem = sems[:_UNROLL]
        ssem = sems[_UNROLL:]
        cid = lax.axis_index("c")
        sid = lax.axis_index("s")
        wid = sid * _NC + cid
        pltpu.sync_copy(zeros_hbm.at[pl.ds(sid * _RPS, _RPS)],
                        acc_sh.at[pl.ds(sid * _RPS, _RPS)])
        plsc.subcore_barrier()

        def body(i, carry):
            j = i * _UNROLL
            gathers = []
            for u in range(_UNROLL):
                gathers.append(pltpu.async_copy(
                    y_hbm.at[src_v.at[j + u]], rows_v.at[u], gsem[u]))
            scatters = []
            for u in range(_UNROLL):
                gathers[u].wait()
                scatters.append(pltpu.async_copy(
                    rows_v.at[u], acc_sh.at[dst_v.at[j + u]], ssem[u], add=True))
            for u in range(_UNROLL):
                scatters[u].wait()
            return carry

        half = _NKC // 2
        for ph in range(2):
            pltpu.sync_copy(srcw.at[wid, pl.ds(ph * half, half)], src_v)
            pltpu.sync_copy(dstw.at[wid, pl.ds(ph * half, half)], dst_v)
            lax.fori_loop(0, half // _UNROLL, body, 0)
        plsc.subcore_barrier()
        pltpu.sync_copy(acc_sh.at[pl.ds(sid * _RPS, _RPS)],
                        out_hbm.at[cid, pl.ds(sid * _RPS, _RPS)])

    return pass_k


def _pad_rows(y):
    return jnp.concatenate([y, jnp.zeros((_NPAD - N, y.shape[1]), y.dtype)], axis=0)


_BLK = 1024

_row_spec = pl.BlockSpec((_BLK, D), lambda i: (i, 0))
_s_spec = pl.BlockSpec((2, _BLK, D), lambda i: (0, i, 0))
_dinv_spec = pl.BlockSpec((_BLK, 1), lambda i: (i, 0))
_vec_spec = pl.BlockSpec((1, D), lambda i: (0, 0))
_w_spec = pl.BlockSpec((D, D), lambda i: (0, 0))


def _scale_mm_kernel(x_ref, w_ref, dinv_ref, o_ref):
    o_ref[...] = dinv_ref[...] * jnp.dot(
        x_ref[...], w_ref[...], preferred_element_type=jnp.float32)


def _scale_mm(x, w, dinvp):
    return pl.pallas_call(
        _scale_mm_kernel,
        grid=(_NPAD // _BLK,),
        in_specs=[_row_spec, _w_spec, _dinv_spec],
        out_specs=_row_spec,
        out_shape=jax.ShapeDtypeStruct((_NPAD, D), jnp.float32),
    )(x, w, dinvp)


def _epi_mm_kernel(s_ref, y_ref, dinv_ref, b_ref, w_ref, o_ref):
    d = dinv_ref[...]
    t = jax.nn.relu(d * (s_ref[0] + s_ref[1] + y_ref[...]) + b_ref[...])
    o_ref[...] = d * jnp.dot(t, w_ref[...], preferred_element_type=jnp.float32)


def _epi_mm(s, y, dinvp, b, w):
    return pl.pallas_call(
        _epi_mm_kernel,
        grid=(_NPAD // _BLK,),
        in_specs=[_s_spec, _row_spec, _dinv_spec, _vec_spec, _w_spec],
        out_specs=_row_spec,
        out_shape=jax.ShapeDtypeStruct((_NPAD, D), jnp.float32),
    )(s, y, dinvp, b, w)


def _epi_mm_bias_kernel(s_ref, y_ref, dinv_ref, b_ref, w_ref, b2_ref, o_ref):
    d = dinv_ref[...]
    t = jax.nn.relu(d * (s_ref[0] + s_ref[1] + y_ref[...]) + b_ref[...])
    o_ref[...] = jnp.dot(t, w_ref[...],
                         preferred_element_type=jnp.float32) + b2_ref[...]


def _epi_mm_bias(s, y, dinvp, b, w, b2):
    return pl.pallas_call(
        _epi_mm_bias_kernel,
        grid=(_NPAD // _BLK,),
        in_specs=[_s_spec, _row_spec, _dinv_spec, _vec_spec, _w_spec, _vec_spec],
        out_specs=_row_spec,
        out_shape=jax.ShapeDtypeStruct((_NPAD, D), jnp.float32),
    )(s, y, dinvp, b, w, b2)


def _epi_kernel(s_ref, y_ref, dinv_ref, b_ref, o_ref):
    o_ref[...] = dinv_ref[...] * (s_ref[0] + s_ref[1] + y_ref[...]) + b_ref[...]


def _epi(s, y, dinvp, b):
    return pl.pallas_call(
        _epi_kernel,
        grid=(_NPAD // _BLK,),
        in_specs=[_s_spec, _row_spec, _dinv_spec, _vec_spec],
        out_specs=_row_spec,
        out_shape=jax.ShapeDtypeStruct((_NPAD, D), jnp.float32),
    )(s, y, dinvp, b)


_SP = 80


def _pma_att_kernel(offs_ref, k_ref, v_ref, qb_ref, o_ref):
    b = pl.program_id(0)
    start = offs_ref[b]
    end = offs_ref[b + 1]
    c0 = start // 128
    c1 = lax.div(end + 127, 128)

    def body(c, carry):
        m, l, accs = carry
        base = c * 128
        rows = k_ref[pl.ds(base * 1, 128), :]
        vrows = v_ref[pl.ds(base * 1, 128), :]
        ids = base + lax.broadcasted_iota(jnp.int32, (128, 1), 0)
        valid = (ids >= start) & (ids < end)
        s = jnp.dot(rows, qb_ref[...], preferred_element_type=jnp.float32)
        s = jnp.where(valid, s, -1e30)
        m_new = jnp.maximum(m, jnp.max(s, axis=0, keepdims=True))
        corr = jnp.exp(m - m_new)
        e = jnp.exp(s - m_new)
        e = jnp.where(valid, e, 0.0)
        l = l * corr + jnp.sum(e, axis=0, keepdims=True)
        new_accs = []
        for h in range(H):
            eh = e[:, h * _SP:(h + 1) * _SP]
            vh = vrows[:, h * DH:(h + 1) * DH]
            ch = corr[0, h * _SP:(h + 1) * _SP]
            prod = lax.dot_general(eh, vh, (((0,), (0,)), ((), ())),
                                   preferred_element_type=jnp.float32)
            new_accs.append(accs[h] * ch[:, None] + prod)
        return m_new, l, tuple(new_accs)

    m0 = jnp.full((1, H * _SP), -1e30, jnp.float32)
    l0 = jnp.zeros((1, H * _SP), jnp.float32)
    a0 = tuple(jnp.zeros((_SP, DH), jnp.float32) for _ in range(H))
    m, l, accs = lax.fori_loop(c0, c1, body, (m0, l0, a0))
    outs = []
    for h in range(H):
        lh = jnp.maximum(l[0, h * _SP:(h + 1) * _SP], 1e-12)
        outs.append(accs[h] / lh[:, None])
    o_ref[0] = jnp.concatenate(outs, axis=1)


def _pma_attention(q, kp, vp, offs):
    qh = q.reshape(K1, H, DH)
    qb = jnp.zeros((H, DH, H, _SP), jnp.float32)
    qb = qb.at[jnp.arange(H), :, jnp.arange(H), :K1].set(
        jnp.transpose(qh, (1, 2, 0)) / np.sqrt(D))
    qb = qb.reshape(D, H * _SP)
    att = pl.pallas_call(
        _pma_att_kernel,
        grid=(B,),
        in_specs=[
            pl.BlockSpec(memory_space=pltpu.SMEM),
            pl.BlockSpec((_NPAD, D), lambda b: (0, 0)),
            pl.BlockSpec((_NPAD, D), lambda b: (0, 0)),
            pl.BlockSpec((D, H * _SP), lambda b: (0, 0)),
        ],
        out_specs=pl.BlockSpec((1, _SP, D), lambda b: (b, 0, 0)),
        out_shape=jax.ShapeDtypeStruct((B, _SP, D), jnp.float32),
    )(offs, kp, vp, qb)
    return att[:, :K1, :]


def _pma_graph(xn, offs, p, dinvp, srcw, dstw, zeros):
    q = p['S1'] @ p['p1_q_w'] + p['p1_q_b']
    scatter = _make_row_scatter(D)
    yk = _scale_mm(xn, p['p1_k_w'], dinvp)
    kp = _epi(scatter(yk, srcw, dstw, zeros), yk, dinvp, p['p1_k_b'].reshape(1, D))
    yv = _scale_mm(xn, p['p1_v_w'], dinvp)
    vp = _epi(scatter(yv, srcw, dstw, zeros), yv, dinvp, p['p1_v_b'].reshape(1, D))
    att = _pma_attention(q, kp, vp, offs)
    o = q[None] + att
    o = o + jax.nn.relu(o @ p['p1_o_w'] + p['p1_o_b'])
    return o


def _mab_dense(q_in, kv_in, p, prefix):
    q = q_in @ p[prefix + '_q_w'] + p[prefix + '_q_b']
    k = kv_in @ p[prefix + '_k_w'] + p[prefix + '_k_b']
    v = kv_in @ p[prefix + '_v_w'] + p[prefix + '_v_b']
    b, s = q.shape[0], q.shape[1]
    n = k.shape[1]
    qh = q.reshape(b, s, H, DH)
    kh = k.reshape(b, n, H, DH)
    vh = v.reshape(b, n, H, DH)
    scores = jnp.einsum('bshd,bnhd->bhsn', qh, kh) / np.sqrt(D)
    a = jax.nn.softmax(scores, axis=-1)
    att = jnp.einsum('bhsn,bnhd->bshd', a, vh).reshape(b, s, D)
    o = q + att
    o = o + jax.nn.relu(o @ p[prefix + '_o_w'] + p[prefix + '_o_b'])
    return o


def _tail_kernel(x_ref, w2_ref, b2_ref, wg_ref, bg_ref, o_ref):
    h = jnp.dot(x_ref[...], w2_ref[...], preferred_element_type=jnp.float32)
    h = h + b2_ref[...]
    g = jnp.dot(h, wg_ref[...], preferred_element_type=jnp.float32) + bg_ref[...]
    o_ref[...] = jax.nn.sigmoid(g)


def kernel(herg_em, x, edge_index, edge_attr, batch, mol_rep, params):
    p = params
    src = edge_index[0].astype(jnp.int32)
    dst = edge_index[1].astype(jnp.int32)
    pad_idx = (N + jnp.arange(_EPAD - E, dtype=jnp.int32) % (_NPAD - N))
    srcw = jnp.concatenate([src, pad_idx]).reshape(_NW, _NKC, 128)
    dstw = jnp.concatenate([dst, pad_idx]).reshape(_NW, _NKC, 128)
    zeros = jnp.zeros((_NPAD, D), jnp.float32)
    ones_table = jnp.concatenate(
        [jnp.ones((N, D), jnp.float32), jnp.zeros((_NPAD - N, D), jnp.float32)], axis=0)

    scatter = _make_row_scatter(D)
    degp = scatter(ones_table, srcw, dstw, zeros)
    deg = degp[0, :N, 0] + degp[1, :N, 0] + 1.0
    dinvp = jnp.concatenate(
        [lax.rsqrt(deg), jnp.zeros(_NPAD - N, jnp.float32)])[:, None]

    y = _scale_mm(_pad_rows(x), p['conv0_w'], dinvp)
    for i in (0, 1, 2):
        s = scatter(y, srcw, dstw, zeros)
        y = _epi_mm(s, y, dinvp, p['conv%d_b' % i].reshape(1, D),
                    p['conv%d_w' % (i + 1)])
    s = scatter(y, srcw, dstw, zeros)
    xn = _epi_mm_bias(s, y, dinvp, p['conv3_b'].reshape(1, D),
                      p['lin1_w'], p['lin1_b'].reshape(1, D))
    offs = jnp.searchsorted(batch.astype(jnp.int32), jnp.arange(B + 1, dtype=jnp.int32)).astype(jnp.int32)
    X = _pma_graph(xn, offs, p, dinvp, srcw, dstw, zeros)
    X = _mab_dense(X, X, p, 'sab')
    X = _mab_dense(jnp.broadcast_to(p['S2'][None], (B, 1, D)), X, p, 'p2')
    X = X[:, 0, :]
    out = pl.pallas_call(
        _tail_kernel,
        out_shape=jax.ShapeDtypeStruct((B, 1), jnp.float32),
    )(X, p['lin2_w'], p['lin2_b'], p['gout_w'], p['gout_b'])
    return out


# speedup vs baseline: 13.5220x; 1.0366x over previous
import functools

import jax
import jax.numpy as jnp
import numpy as np
from jax import lax
from jax.experimental import pallas as pl
from jax.experimental.pallas import tpu as pltpu
from jax.experimental.pallas import tpu_sc as plsc

N = 10000
E = 320000
D = 128
B = 256
H = 8
DH = D // H
K1 = 75

_NC, _NS = 2, 16
_NW = _NC * _NS
_NPAD = N + 240
_RPS = _NPAD // _NS
_NKC = 80
_EPAD = _NW * _NKC * 128

_UNROLL = 2


@functools.lru_cache(maxsize=None)
def _make_row_scatter(dcols):
    mesh = plsc.VectorSubcoreMesh(core_axis_name="c", subcore_axis_name="s")

    @functools.partial(
        pl.kernel,
        out_type=jax.ShapeDtypeStruct((_NC, _NPAD, dcols), jnp.float32),
        mesh=mesh,
        scratch_types=[
            pltpu.VMEM((_NKC // 2, 128), jnp.int32),
            pltpu.VMEM((_NKC // 2, 128), jnp.int32),
            pltpu.VMEM((_UNROLL, 128, dcols), jnp.float32),
            pltpu.VMEM_SHARED((_NPAD, dcols), jnp.float32),
        ] + [pltpu.SemaphoreType.DMA] * (2 * _UNROLL),
    )
    def pass_k(y_hbm, srcw, dstw, zeros_hbm, out_hbm, src_v, dst_v, rows_v, acc_sh, *sems):
        gsem = sems[:_UNROLL]
        ssem = sems[_UNROLL:]
        cid = lax.axis_index("c")
        sid = lax.axis_index("s")
        wid = sid * _NC + cid
        pltpu.sync_copy(zeros_hbm.at[pl.ds(sid * _RPS, _RPS)],
                        acc_sh.at[pl.ds(sid * _RPS, _RPS)])
        plsc.subcore_barrier()

        def body(i, carry):
            j = i * _UNROLL
            gathers = []
            for u in range(_UNROLL):
                gathers.append(pltpu.async_copy(
                    y_hbm.at[src_v.at[j + u]], rows_v.at[u], gsem[u]))
            scatters = []
            for u in range(_UNROLL):
                gathers[u].wait()
                scatters.append(pltpu.async_copy(
                    rows_v.at[u], acc_sh.at[dst_v.at[j + u]], ssem[u], add=True))
            for u in range(_UNROLL):
                scatters[u].wait()
            return carry

        half = _NKC // 2
        for ph in range(2):
            pltpu.sync_copy(srcw.at[wid, pl.ds(ph * half, half)], src_v)
            pltpu.sync_copy(dstw.at[wid, pl.ds(ph * half, half)], dst_v)
            lax.fori_loop(0, half // _UNROLL, body, 0)
        plsc.subcore_barrier()
        pltpu.sync_copy(acc_sh.at[pl.ds(sid * _RPS, _RPS)],
                        out_hbm.at[cid, pl.ds(sid * _RPS, _RPS)])

    return pass_k


def _pad_rows(y):
    return jnp.concatenate([y, jnp.zeros((_NPAD - N, y.shape[1]), y.dtype)], axis=0)


_BLK = 1024

_row_spec = pl.BlockSpec((_BLK, D), lambda i: (i, 0))
_s_spec = pl.BlockSpec((2, _BLK, D), lambda i: (0, i, 0))
_dinv_spec = pl.BlockSpec((_BLK, 1), lambda i: (i, 0))
_vec_spec = pl.BlockSpec((1, D), lambda i: (0, 0))
_w_spec = pl.BlockSpec((D, D), lambda i: (0, 0))


def _scale_mm_kernel(x_ref, w_ref, dinv_ref, o_ref):
    o_ref[...] = dinv_ref[...] * jnp.dot(
        x_ref[...], w_ref[...], preferred_element_type=jnp.float32)


def _scale_mm(x, w, dinvp):
    return pl.pallas_call(
        _scale_mm_kernel,
        grid=(_NPAD // _BLK,),
        in_specs=[_row_spec, _w_spec, _dinv_spec],
        out_specs=_row_spec,
        out_shape=jax.ShapeDtypeStruct((_NPAD, D), jnp.float32),
    )(x, w, dinvp)


def _epi_mm_kernel(s_ref, y_ref, dinv_ref, b_ref, w_ref, o_ref):
    d = dinv_ref[...]
    t = jax.nn.relu(d * (s_ref[0] + s_ref[1] + y_ref[...]) + b_ref[...])
    o_ref[...] = d * jnp.dot(t, w_ref[...], preferred_element_type=jnp.float32)


def _epi_mm(s, y, dinvp, b, w):
    return pl.pallas_call(
        _epi_mm_kernel,
        grid=(_NPAD // _BLK,),
        in_specs=[_s_spec, _row_spec, _dinv_spec, _vec_spec, _w_spec],
        out_specs=_row_spec,
        out_shape=jax.ShapeDtypeStruct((_NPAD, D), jnp.float32),
    )(s, y, dinvp, b, w)


def _epi_mm_bias_kernel(s_ref, y_ref, dinv_ref, b_ref, w_ref, b2_ref, o_ref):
    d = dinv_ref[...]
    t = jax.nn.relu(d * (s_ref[0] + s_ref[1] + y_ref[...]) + b_ref[...])
    o_ref[...] = jnp.dot(t, w_ref[...],
                         preferred_element_type=jnp.float32) + b2_ref[...]


def _epi_mm_bias(s, y, dinvp, b, w, b2):
    return pl.pallas_call(
        _epi_mm_bias_kernel,
        grid=(_NPAD // _BLK,),
        in_specs=[_s_spec, _row_spec, _dinv_spec, _vec_spec, _w_spec, _vec_spec],
        out_specs=_row_spec,
        out_shape=jax.ShapeDtypeStruct((_NPAD, D), jnp.float32),
    )(s, y, dinvp, b, w, b2)


def _epi_kernel(s_ref, y_ref, dinv_ref, b_ref, o_ref):
    o_ref[...] = dinv_ref[...] * (s_ref[0] + s_ref[1] + y_ref[...]) + b_ref[...]


def _epi(s, y, dinvp, b):
    return pl.pallas_call(
        _epi_kernel,
        grid=(_NPAD // _BLK,),
        in_specs=[_s_spec, _row_spec, _dinv_spec, _vec_spec],
        out_specs=_row_spec,
        out_shape=jax.ShapeDtypeStruct((_NPAD, D), jnp.float32),
    )(s, y, dinvp, b)


_SP = 80


def _pma_att_kernel(offs_ref, k_ref, v_ref, qb_ref, o_ref):
    b = pl.program_id(0)
    start = offs_ref[b]
    end = offs_ref[b + 1]
    c0 = start // 128
    c1 = lax.div(end + 127, 128)

    def body(c, carry):
        m, l, acc = carry
        base = c * 128
        rows = k_ref[pl.ds(base * 1, 128), :]
        vrows = v_ref[pl.ds(base * 1, 128), :]
        ids = base + lax.broadcasted_iota(jnp.int32, (128, 1), 0)
        valid = (ids >= start) & (ids < end)
        s = jnp.dot(rows, qb_ref[...], preferred_element_type=jnp.float32)
        s = jnp.where(valid, s, -1e30)
        m_new = jnp.maximum(m, jnp.max(s, axis=0, keepdims=True))
        corr = jnp.exp(m - m_new)
        e = jnp.exp(s - m_new)
        e = jnp.where(valid, e, 0.0)
        l = l * corr + jnp.sum(e, axis=0, keepdims=True)
        full = lax.dot_general(vrows, e, (((0,), (0,)), ((), ())),
                               preferred_element_type=jnp.float32)
        ext = jnp.concatenate(
            [full[h * DH:(h + 1) * DH, h * _SP:(h + 1) * _SP] for h in range(H)],
            axis=1)
        acc = acc * corr + ext
        return m_new, l, acc

    m0 = jnp.full((1, H * _SP), -1e30, jnp.float32)
    l0 = jnp.zeros((1, H * _SP), jnp.float32)
    a0 = jnp.zeros((DH, H * _SP), jnp.float32)
    m, l, acc = lax.fori_loop(c0, c1, body, (m0, l0, a0))
    acc = acc / jnp.maximum(l, 1e-12)
    o_ref[0] = jnp.concatenate(
        [jnp.transpose(acc[:, h * _SP:(h + 1) * _SP]) for h in range(H)], axis=1)


def _pma_attention(q, kp, vp, offs):
    qh = q.reshape(K1, H, DH)
    qb = jnp.zeros((H, DH, H, _SP), jnp.float32)
    qb = qb.at[jnp.arange(H), :, jnp.arange(H), :K1].set(
        jnp.transpose(qh, (1, 2, 0)) / np.sqrt(D))
    qb = qb.reshape(D, H * _SP)
    att = pl.pallas_call(
        _pma_att_kernel,
        grid=(B,),
        in_specs=[
            pl.BlockSpec(memory_space=pltpu.SMEM),
            pl.BlockSpec((_NPAD, D), lambda b: (0, 0)),
            pl.BlockSpec((_NPAD, D), lambda b: (0, 0)),
            pl.BlockSpec((D, H * _SP), lambda b: (0, 0)),
        ],
        out_specs=pl.BlockSpec((1, _SP, D), lambda b: (b, 0, 0)),
        out_shape=jax.ShapeDtypeStruct((B, _SP, D), jnp.float32),
    )(offs, kp, vp, qb)
    return att[:, :K1, :]


def _pma_graph(xn, offs, p, dinvp, srcw, dstw, zeros):
    q = p['S1'] @ p['p1_q_w'] + p['p1_q_b']
    scatter = _make_row_scatter(D)
    yk = _scale_mm(xn, p['p1_k_w'], dinvp)
    kp = _epi(scatter(yk, srcw, dstw, zeros), yk, dinvp, p['p1_k_b'].reshape(1, D))
    yv = _scale_mm(xn, p['p1_v_w'], dinvp)
    vp = _epi(scatter(yv, srcw, dstw, zeros), yv, dinvp, p['p1_v_b'].reshape(1, D))
    att = _pma_attention(q, kp, vp, offs)
    o = q[None] + att
    o = o + jax.nn.relu(o @ p['p1_o_w'] + p['p1_o_b'])
    return o


def _mab_dense(q_in, kv_in, p, prefix):
    q = q_in @ p[prefix + '_q_w'] + p[prefix + '_q_b']
    k = kv_in @ p[prefix + '_k_w'] + p[prefix + '_k_b']
    v = kv_in @ p[prefix + '_v_w'] + p[prefix + '_v_b']
    b, s = q.shape[0], q.shape[1]
    n = k.shape[1]
    qh = q.reshape(b, s, H, DH)
    kh = k.reshape(b, n, H, DH)
    vh = v.reshape(b, n, H, DH)
    scores = jnp.einsum('bshd,bnhd->bhsn', qh, kh) / np.sqrt(D)
    a = jax.nn.softmax(scores, axis=-1)
    att = jnp.einsum('bhsn,bnhd->bshd', a, vh).reshape(b, s, D)
    o = q + att
    o = o + jax.nn.relu(o @ p[prefix + '_o_w'] + p[prefix + '_o_b'])
    return o


def _tail_kernel(x_ref, w2_ref, b2_ref, wg_ref, bg_ref, o_ref):
    h = jnp.dot(x_ref[...], w2_ref[...], preferred_element_type=jnp.float32)
    h = h + b2_ref[...]
    g = jnp.dot(h, wg_ref[...], preferred_element_type=jnp.float32) + bg_ref[...]
    o_ref[...] = jax.nn.sigmoid(g)


def kernel(herg_em, x, edge_index, edge_attr, batch, mol_rep, params):
    p = params
    src = edge_index[0].astype(jnp.int32)
    dst = edge_index[1].astype(jnp.int32)
    pad_idx = (N + jnp.arange(_EPAD - E, dtype=jnp.int32) % (_NPAD - N))
    srcw = jnp.concatenate([src, pad_idx]).reshape(_NW, _NKC, 128)
    dstw = jnp.concatenate([dst, pad_idx]).reshape(_NW, _NKC, 128)
    zeros = jnp.zeros((_NPAD, D), jnp.float32)
    ones_table = jnp.concatenate(
        [jnp.ones((N, D), jnp.float32), jnp.zeros((_NPAD - N, D), jnp.float32)], axis=0)

    scatter = _make_row_scatter(D)
    degp = scatter(ones_table, srcw, dstw, zeros)
    deg = degp[0, :N, 0] + degp[1, :N, 0] + 1.0
    dinvp = jnp.concatenate(
        [lax.rsqrt(deg), jnp.zeros(_NPAD - N, jnp.float32)])[:, None]

    y = _scale_mm(_pad_rows(x), p['conv0_w'], dinvp)
    for i in (0, 1, 2):
        s = scatter(y, srcw, dstw, zeros)
        y = _epi_mm(s, y, dinvp, p['conv%d_b' % i].reshape(1, D),
                    p['conv%d_w' % (i + 1)])
    s = scatter(y, srcw, dstw, zeros)
    xn = _epi_mm_bias(s, y, dinvp, p['conv3_b'].reshape(1, D),
                      p['lin1_w'], p['lin1_b'].reshape(1, D))
    offs = jnp.searchsorted(batch.astype(jnp.int32), jnp.arange(B + 1, dtype=jnp.int32)).astype(jnp.int32)
    X = _pma_graph(xn, offs, p, dinvp, srcw, dstw, zeros)
    X = _mab_dense(X, X, p, 'sab')
    X = _mab_dense(jnp.broadcast_to(p['S2'][None], (B, 1, D)), X, p, 'p2')
    X = X[:, 0, :]
    out = pl.pallas_call(
        _tail_kernel,
        out_shape=jax.ShapeDtypeStruct((B, 1), jnp.float32),
    )(X, p['lin2_w'], p['lin2_b'], p['gout_w'], p['gout_b'])
    return out
